# baseline (speedup 1.0000x reference)
import functools

import jax
import jax.numpy as jnp
from jax.experimental import pallas as pl

_KT = 1024


def _mm_kernel(wt_ref, xt_ref, b_ref, o_ref, *, nsteps, k_total):
    k = pl.program_id(0)

    @pl.when(k == 0)
    def _():
        o_ref[...] = jnp.broadcast_to(b_ref[...], o_ref.shape)

    @pl.when(k < nsteps - 1)
    def _():
        o_ref[...] += jnp.dot(
            wt_ref[...], xt_ref[...], preferred_element_type=jnp.float32
        )

    @pl.when(k == nsteps - 1)
    def _():
        valid = k_total - k * _KT
        wt = wt_ref[...]
        xt = xt_ref[...]
        wcol = jax.lax.broadcasted_iota(jnp.int32, wt.shape, 1)
        xrow = jax.lax.broadcasted_iota(jnp.int32, xt.shape, 0)
        wt = jnp.where(wcol < valid, wt, 0.0)
        xt = jnp.where(xrow < valid, xt, 0.0)
        o_ref[...] += jnp.dot(wt, xt, preferred_element_type=jnp.float32)


def kernel(X, W, b):
    M, K = X.shape
    N = W.shape[1]
    XT = X.T
    WT = W.T
    nsteps = pl.cdiv(K, _KT)
    b2 = b.reshape(N, 1)
    outT = pl.pallas_call(
        functools.partial(_mm_kernel, nsteps=nsteps, k_total=K),
        grid=(nsteps,),
        in_specs=[
            pl.BlockSpec((N, _KT), lambda k: (0, k)),
            pl.BlockSpec((_KT, M), lambda k: (k, 0)),
            pl.BlockSpec((N, 1), lambda k: (0, 0)),
        ],
        out_specs=pl.BlockSpec((N, M), lambda k: (0, 0)),
        out_shape=jax.ShapeDtypeStruct((N, M), jnp.float32),
    )(WT, XT, b2)
    return outT.T


# speedup vs baseline: 1.1194x; 1.1194x over previous
import functools

import jax
import jax.numpy as jnp
from jax.experimental import pallas as pl

_KT = 2048


def _mm_kernel(wt_ref, xt_ref, b_ref, o_ref, *, nsteps, k_total):
    k = pl.program_id(0)

    @pl.when(k == 0)
    def _():
        o_ref[...] = jnp.broadcast_to(b_ref[...], o_ref.shape)

    @pl.when(k < nsteps - 1)
    def _():
        o_ref[...] += jnp.dot(
            wt_ref[...], xt_ref[...], preferred_element_type=jnp.float32
        )

    @pl.when(k == nsteps - 1)
    def _():
        valid = k_total - k * _KT
        wt = wt_ref[...]
        xt = xt_ref[...]
        wcol = jax.lax.broadcasted_iota(jnp.int32, wt.shape, 1)
        xrow = jax.lax.broadcasted_iota(jnp.int32, xt.shape, 0)
        wt = jnp.where(wcol < valid, wt, 0.0)
        xt = jnp.where(xrow < valid, xt, 0.0)
        o_ref[...] += jnp.dot(wt, xt, preferred_element_type=jnp.float32)


def kernel(X, W, b):
    M, K = X.shape
    N = W.shape[1]
    XT = X.T
    WT = W.T
    nsteps = pl.cdiv(K, _KT)
    b2 = b.reshape(N, 1)
    outT = pl.pallas_call(
        functools.partial(_mm_kernel, nsteps=nsteps, k_total=K),
        grid=(nsteps,),
        in_specs=[
            pl.BlockSpec((N, _KT), lambda k: (0, k)),
            pl.BlockSpec((_KT, M), lambda k: (k, 0)),
            pl.BlockSpec((N, 1), lambda k: (0, 0)),
        ],
        out_specs=pl.BlockSpec((N, M), lambda k: (0, 0)),
        out_shape=jax.ShapeDtypeStruct((N, M), jnp.float32),
    )(WT, XT, b2)
    return outT.T


# speedup vs baseline: 1.1464x; 1.0242x over previous
import functools

import jax
import jax.numpy as jnp
from jax.experimental import pallas as pl

_KT = 2560


def _mm_kernel(wt_ref, xt_ref, b_ref, o_ref, *, nsteps, k_total):
    k = pl.program_id(0)

    @pl.when(k == 0)
    def _():
        o_ref[...] = jnp.broadcast_to(b_ref[...], o_ref.shape)

    @pl.when(k < nsteps - 1)
    def _():
        o_ref[...] += jnp.dot(
            wt_ref[...], xt_ref[...], preferred_element_type=jnp.float32
        )

    @pl.when(k == nsteps - 1)
    def _():
        valid = k_total - k * _KT
        wt = wt_ref[...]
        xt = xt_ref[...]
        wcol = jax.lax.broadcasted_iota(jnp.int32, wt.shape, 1)
        xrow = jax.lax.broadcasted_iota(jnp.int32, xt.shape, 0)
        wt = jnp.where(wcol < valid, wt, 0.0)
        xt = jnp.where(xrow < valid, xt, 0.0)
        o_ref[...] += jnp.dot(wt, xt, preferred_element_type=jnp.float32)


def kernel(X, W, b):
    M, K = X.shape
    N = W.shape[1]
    XT = X.T
    WT = W.T
    nsteps = pl.cdiv(K, _KT)
    b2 = b.reshape(N, 1)
    outT = pl.pallas_call(
        functools.partial(_mm_kernel, nsteps=nsteps, k_total=K),
        grid=(nsteps,),
        in_specs=[
            pl.BlockSpec((N, _KT), lambda k: (0, k)),
            pl.BlockSpec((_KT, M), lambda k: (k, 0)),
            pl.BlockSpec((N, 1), lambda k: (0, 0)),
        ],
        out_specs=pl.BlockSpec((N, M), lambda k: (0, 0)),
        out_shape=jax.ShapeDtypeStruct((N, M), jnp.float32),
    )(WT, XT, b2)
    return outT.T
